# Initial kernel scaffold; baseline (speedup 1.0000x reference)
import jax
import jax.numpy as jnp
from jax.experimental import pallas as pl


def kernel(x, edge_index, Ws0, Wn0, b0, Ws1, Wn1, b1, Ws2, Wn2, b2):
    raise NotImplementedError("write your pallas kernel here")


# speedup vs baseline: 6.7237x; 6.7237x over previous
import functools

import jax
import jax.numpy as jnp
from jax import lax
from jax.experimental import pallas as pl
from jax.experimental.pallas import tpu as pltpu
from jax.experimental.pallas import tpu_sc as plsc

N_NODES = 10000
N_EDGES = 160000
IN_FEATS = 256
N_HIDDEN = 512
N_CLASSES = 64

NC = 2
NS = 16
NW = NC * NS
EPW = N_EDGES // NW
B = 50
IT = EPW // B
RPT = 624
RZ = 48
REM = N_NODES - NS * RPT

M_BLK = 1000
G = N_NODES // M_BLK


_MESH = plsc.VectorSubcoreMesh(core_axis_name="c", subcore_axis_name="s")
_REM0 = NS * RPT


def _make_sc_agg(n_chunks, feat):
  per_row = feat // 16

  out_type = [jax.ShapeDtypeStruct((NC, N_NODES, feat), jnp.float32)
              for _ in range(n_chunks)]
  scratch = [
      pltpu.VMEM((IT, B), jnp.int32),
      pltpu.VMEM((IT, B), jnp.int32),
      pltpu.VMEM((B, feat), jnp.float32),
      pltpu.VMEM((B, feat), jnp.float32),
      pltpu.VMEM((RZ, feat), jnp.float32),
      pltpu.VMEM_SHARED((N_NODES, feat), jnp.float32),
      pltpu.SemaphoreType.DMA,
      pltpu.SemaphoreType.DMA,
  ]

  def body(*refs):
    x_refs = refs[:n_chunks]
    src_hbm, dst_hbm = refs[n_chunks], refs[n_chunks + 1]
    o = n_chunks + 2
    out_refs = refs[o:o + n_chunks]
    src_v, dst_v, rows0, rows1, zbuf, acc, sem0, sem1 = refs[o + n_chunks:]

    cid = lax.axis_index("c")
    sid = lax.axis_index("s")
    wid = sid * NC + cid

    pltpu.sync_copy(src_hbm.at[wid], src_v)
    pltpu.sync_copy(dst_hbm.at[wid], dst_v)

    @pl.loop(0, RZ * per_row)
    def _fill_z(i):
      zbuf[i // per_row, pl.ds((i % per_row) * 16, 16)] = jnp.zeros(
          (16,), jnp.float32)

    row0 = sid * RPT

    def process_chunk(x_ref):
      pltpu.async_copy(x_ref.at[src_v.at[0]], rows0, sem0)
      pltpu.async_copy(x_ref.at[src_v.at[1]], rows1, sem1)

      @pl.loop(0, IT, step=2)
      def _edges(j):
        pltpu.make_async_copy(x_ref.at[src_v.at[j]], rows0, sem0).wait()
        pltpu.sync_copy(rows0, acc.at[dst_v.at[j]], add=True)

        @pl.when(j + 2 < IT)
        def _():
          pltpu.async_copy(x_ref.at[src_v.at[j + 2]], rows0, sem0)

        pltpu.make_async_copy(x_ref.at[src_v.at[j + 1]], rows1, sem1).wait()
        pltpu.sync_copy(rows1, acc.at[dst_v.at[j + 1]], add=True)

        @pl.when(j + 3 < IT)
        def _():
          pltpu.async_copy(x_ref.at[src_v.at[j + 3]], rows1, sem1)

    for ci in range(n_chunks):
      for k in range(RPT // RZ):
        pltpu.sync_copy(zbuf, acc.at[pl.ds(row0 + k * RZ, RZ)])

      @pl.when(sid == NS - 1)
      def _zero_rem():
        pltpu.sync_copy(zbuf.at[pl.ds(0, REM)], acc.at[pl.ds(_REM0, REM)])

      plsc.subcore_barrier()
      process_chunk(x_refs[ci])
      plsc.subcore_barrier()
      pltpu.sync_copy(acc.at[pl.ds(row0, RPT)],
                      out_refs[ci].at[cid, pl.ds(row0, RPT)])

      @pl.when(sid == NS - 1)
      def _out_rem():
        pltpu.sync_copy(acc.at[pl.ds(_REM0, REM)],
                        out_refs[ci].at[cid, pl.ds(_REM0, REM)])

      if ci + 1 < n_chunks:
        plsc.subcore_barrier()

  return pl.kernel(body, out_type=out_type, mesh=_MESH, scratch_types=scratch)


def _sc_deg_body(dst_hbm, deg_out, dst_v, ones_v, zdeg, degacc):
  cid = lax.axis_index("c")
  sid = lax.axis_index("s")
  wid = sid * NC + cid

  pltpu.sync_copy(dst_hbm.at[wid], dst_v)

  @pl.loop(0, B * 8)
  def _fill_o(i):
    ones_v[i // 8, pl.ds((i % 8) * 16, 16)] = jnp.full((16,), 1.0, jnp.float32)

  @pl.loop(0, RZ * 8)
  def _fill_zd(i):
    zdeg[i // 8, pl.ds((i % 8) * 16, 16)] = jnp.zeros((16,), jnp.float32)

  row0 = sid * RPT
  for k in range(RPT // RZ):
    pltpu.sync_copy(zdeg, degacc.at[pl.ds(row0 + k * RZ, RZ)])

  @pl.when(sid == NS - 1)
  def _zero_rem():
    pltpu.sync_copy(zdeg.at[pl.ds(0, REM)], degacc.at[pl.ds(_REM0, REM)])

  plsc.subcore_barrier()

  @pl.loop(0, IT)
  def _edges(j):
    pltpu.sync_copy(ones_v, degacc.at[dst_v.at[j]], add=True)

  plsc.subcore_barrier()
  pltpu.sync_copy(degacc.at[pl.ds(row0, RPT)],
                  deg_out.at[cid, pl.ds(row0, RPT)])

  @pl.when(sid == NS - 1)
  def _out_rem():
    pltpu.sync_copy(degacc.at[pl.ds(_REM0, REM)],
                    deg_out.at[cid, pl.ds(_REM0, REM)])


_sc_deg = pl.kernel(
    _sc_deg_body,
    out_type=jax.ShapeDtypeStruct((NC, N_NODES, 128), jnp.float32),
    mesh=_MESH,
    scratch_types=[
        pltpu.VMEM((IT, B), jnp.int32),
        pltpu.VMEM((B, 128), jnp.float32),
        pltpu.VMEM((RZ, 128), jnp.float32),
        pltpu.VMEM_SHARED((N_NODES, 128), jnp.float32),
    ])


def _inv_deg(dg):
  deg = dg[0, :, 0] + dg[1, :, 0]
  return (1.0 / jnp.maximum(deg, 1.0))[:, None]


def _dot(a, b):
  return jnp.dot(a, b, preferred_element_type=jnp.float32)


def _mm0_body(x0, x1, a0, a1, dg, ws, wn, b, o0, o1, o2, o3):
  inv = _inv_deg(dg[...])
  h = (_dot(x0[...], ws[0:128, :]) + _dot(x1[...], ws[128:256, :])
       + _dot((a0[0] + a0[1]) * inv, wn[0:128, :])
       + _dot((a1[0] + a1[1]) * inv, wn[128:256, :])
       + b[...][None, :])
  h = jnp.maximum(h, 0.0)
  o0[...] = h[:, 0:128]
  o1[...] = h[:, 128:256]
  o2[...] = h[:, 256:384]
  o3[...] = h[:, 384:512]


def _mm1_body(h0, h1, h2, h3, a0, a1, a2, a3, dg, ws, wn, b, wn2, oh, oz):
  inv = _inv_deg(dg[...])
  acc = b[...][None, :]
  hs = (h0, h1, h2, h3)
  As = (a0, a1, a2, a3)
  for c in range(4):
    acc = acc + _dot(hs[c][...], ws[c * 128:(c + 1) * 128, :])
    acc = acc + _dot((As[c][0] + As[c][1]) * inv, wn[c * 128:(c + 1) * 128, :])
  acc = jnp.maximum(acc, 0.0)
  oh[...] = acc
  z = _dot(acc, wn2[...])
  oz[...] = jnp.concatenate(
      [z, jnp.zeros((z.shape[0], 64), jnp.float32)], axis=1)


def _mm2_body(h, az, dg, ws, b, out):
  inv = _inv_deg(dg[...])
  out[...] = (_dot(h[...], ws[...]) + (az[0, :, 0:64] + az[1, :, 0:64]) * inv
              + b[...][None, :])


def _blk(shape, imap):
  return pl.BlockSpec(shape, imap)


_row = lambda i: (i, 0)
_part = lambda i: (0, i, 0)
_whole2 = lambda i: (0, 0)
_whole1 = lambda i: (0,)


def _mm0(x0, x1, a0, a1, dg, ws, wn, b):
  return pl.pallas_call(
      _mm0_body,
      grid=(G,),
      in_specs=[
          _blk((M_BLK, 128), _row), _blk((M_BLK, 128), _row),
          _blk((NC, M_BLK, 128), _part), _blk((NC, M_BLK, 128), _part),
          _blk((NC, M_BLK, 128), _part),
          _blk((256, 512), _whole2), _blk((256, 512), _whole2),
          _blk((512,), _whole1),
      ],
      out_specs=[_blk((M_BLK, 128), _row)] * 4,
      out_shape=[jax.ShapeDtypeStruct((N_NODES, 128), jnp.float32)] * 4,
  )(x0, x1, a0, a1, dg, ws, wn, b)


def _mm1(hs, As, dg, ws, wn, b, wn2):
  return pl.pallas_call(
      _mm1_body,
      grid=(G,),
      in_specs=(
          [_blk((M_BLK, 128), _row)] * 4
          + [_blk((NC, M_BLK, 128), _part)] * 4
          + [_blk((NC, M_BLK, 128), _part),
             _blk((512, 512), _whole2), _blk((512, 512), _whole2),
             _blk((512,), _whole1), _blk((512, 64), _whole2)]),
      out_specs=[_blk((M_BLK, 512), _row), _blk((M_BLK, 128), _row)],
      out_shape=[jax.ShapeDtypeStruct((N_NODES, 512), jnp.float32),
                 jax.ShapeDtypeStruct((N_NODES, 128), jnp.float32)],
  )(*hs, *As, dg, ws, wn, b, wn2)


def _mm2(h, az, dg, ws, b):
  return pl.pallas_call(
      _mm2_body,
      grid=(G,),
      in_specs=[
          _blk((M_BLK, 512), _row), _blk((NC, M_BLK, 128), _part),
          _blk((NC, M_BLK, 128), _part),
          _blk((512, 64), _whole2), _blk((64,), _whole1),
      ],
      out_specs=_blk((M_BLK, 64), _row),
      out_shape=jax.ShapeDtypeStruct((N_NODES, 64), jnp.float32),
  )(h, az, dg, ws, b)


_sc_agg2 = _make_sc_agg(2, 128)
_sc_agg4 = _make_sc_agg(4, 128)
_sc_agg1 = _make_sc_agg(1, 128)


@jax.jit
def kernel(x, edge_index, Ws0, Wn0, b0, Ws1, Wn1, b1, Ws2, Wn2, b2):
  ei = edge_index.astype(jnp.int32)
  src = ei[0].reshape(NW, IT, B)
  dst = ei[1].reshape(NW, IT, B)

  x0 = x[:, 0:128]
  x1 = x[:, 128:256]

  dg = _sc_deg(dst)
  a0, a1 = _sc_agg2(x0, x1, src, dst)
  hs = _mm0(x0, x1, a0, a1, dg, Ws0, Wn0, b0)
  As = _sc_agg4(*hs, src, dst)
  h2, z = _mm1(hs, As, dg, Ws1, Wn1, b1, Wn2)
  (az,) = _sc_agg1(z, src, dst)
  return _mm2(h2, az, dg, Ws2, b2)


# speedup vs baseline: 7.9875x; 1.1880x over previous
import functools

import jax
import jax.numpy as jnp
from jax import lax
from jax.experimental import pallas as pl
from jax.experimental.pallas import tpu as pltpu
from jax.experimental.pallas import tpu_sc as plsc

N_NODES = 10000
N_EDGES = 160000
IN_FEATS = 256
N_HIDDEN = 512
N_CLASSES = 64

NC = 2
NS = 16
NW = NC * NS
EPW = N_EDGES // NW
B = 100
IT = EPW // B
RPT = 624
RZ = 48
REM = N_NODES - NS * RPT

M_BLK = 1000
G = N_NODES // M_BLK


_MESH = plsc.VectorSubcoreMesh(core_axis_name="c", subcore_axis_name="s")
_REM0 = NS * RPT


def _make_sc_agg(n_chunks, feat):
  per_row = feat // 16

  out_type = [jax.ShapeDtypeStruct((NC, N_NODES, feat), jnp.float32)
              for _ in range(n_chunks)]
  scratch = [
      pltpu.VMEM((IT, B), jnp.int32),
      pltpu.VMEM((IT, B), jnp.int32),
      pltpu.VMEM((B, feat), jnp.float32),
      pltpu.VMEM((B, feat), jnp.float32),
      pltpu.VMEM((RZ, feat), jnp.float32),
      pltpu.VMEM_SHARED((N_NODES, feat), jnp.float32),
      pltpu.SemaphoreType.DMA,
      pltpu.SemaphoreType.DMA,
  ]

  def body(*refs):
    x_refs = refs[:n_chunks]
    src_hbm, dst_hbm = refs[n_chunks], refs[n_chunks + 1]
    o = n_chunks + 2
    out_refs = refs[o:o + n_chunks]
    src_v, dst_v, rows0, rows1, zbuf, acc, sem0, sem1 = refs[o + n_chunks:]

    cid = lax.axis_index("c")
    sid = lax.axis_index("s")
    wid = sid * NC + cid

    pltpu.sync_copy(src_hbm.at[wid], src_v)
    pltpu.sync_copy(dst_hbm.at[wid], dst_v)

    @pl.loop(0, RZ * per_row)
    def _fill_z(i):
      zbuf[i // per_row, pl.ds((i % per_row) * 16, 16)] = jnp.zeros(
          (16,), jnp.float32)

    row0 = sid * RPT

    def process_chunk(x_ref):
      pltpu.async_copy(x_ref.at[src_v.at[0]], rows0, sem0)
      pltpu.async_copy(x_ref.at[src_v.at[1]], rows1, sem1)

      @pl.loop(0, IT, step=2)
      def _edges(j):
        pltpu.make_async_copy(x_ref.at[src_v.at[j]], rows0, sem0).wait()
        pltpu.sync_copy(rows0, acc.at[dst_v.at[j]], add=True)

        @pl.when(j + 2 < IT)
        def _():
          pltpu.async_copy(x_ref.at[src_v.at[j + 2]], rows0, sem0)

        pltpu.make_async_copy(x_ref.at[src_v.at[j + 1]], rows1, sem1).wait()
        pltpu.sync_copy(rows1, acc.at[dst_v.at[j + 1]], add=True)

        @pl.when(j + 3 < IT)
        def _():
          pltpu.async_copy(x_ref.at[src_v.at[j + 3]], rows1, sem1)

    for ci in range(n_chunks):
      for k in range(RPT // RZ):
        pltpu.sync_copy(zbuf, acc.at[pl.ds(row0 + k * RZ, RZ)])

      @pl.when(sid == NS - 1)
      def _zero_rem():
        pltpu.sync_copy(zbuf.at[pl.ds(0, REM)], acc.at[pl.ds(_REM0, REM)])

      plsc.subcore_barrier()
      process_chunk(x_refs[ci])
      plsc.subcore_barrier()
      pltpu.sync_copy(acc.at[pl.ds(row0, RPT)],
                      out_refs[ci].at[cid, pl.ds(row0, RPT)])

      @pl.when(sid == NS - 1)
      def _out_rem():
        pltpu.sync_copy(acc.at[pl.ds(_REM0, REM)],
                        out_refs[ci].at[cid, pl.ds(_REM0, REM)])

      if ci + 1 < n_chunks:
        plsc.subcore_barrier()

  return pl.kernel(body, out_type=out_type, mesh=_MESH, scratch_types=scratch)


def _sc_deg_body(dst_hbm, deg_out, dst_v, ones_v, zdeg, degacc):
  cid = lax.axis_index("c")
  sid = lax.axis_index("s")
  wid = sid * NC + cid

  pltpu.sync_copy(dst_hbm.at[wid], dst_v)

  @pl.loop(0, B * 8)
  def _fill_o(i):
    ones_v[i // 8, pl.ds((i % 8) * 16, 16)] = jnp.full((16,), 1.0, jnp.float32)

  @pl.loop(0, RZ * 8)
  def _fill_zd(i):
    zdeg[i // 8, pl.ds((i % 8) * 16, 16)] = jnp.zeros((16,), jnp.float32)

  row0 = sid * RPT
  for k in range(RPT // RZ):
    pltpu.sync_copy(zdeg, degacc.at[pl.ds(row0 + k * RZ, RZ)])

  @pl.when(sid == NS - 1)
  def _zero_rem():
    pltpu.sync_copy(zdeg.at[pl.ds(0, REM)], degacc.at[pl.ds(_REM0, REM)])

  plsc.subcore_barrier()

  @pl.loop(0, IT)
  def _edges(j):
    pltpu.sync_copy(ones_v, degacc.at[dst_v.at[j]], add=True)

  plsc.subcore_barrier()
  pltpu.sync_copy(degacc.at[pl.ds(row0, RPT)],
                  deg_out.at[cid, pl.ds(row0, RPT)])

  @pl.when(sid == NS - 1)
  def _out_rem():
    pltpu.sync_copy(degacc.at[pl.ds(_REM0, REM)],
                    deg_out.at[cid, pl.ds(_REM0, REM)])


_sc_deg = pl.kernel(
    _sc_deg_body,
    out_type=jax.ShapeDtypeStruct((NC, N_NODES, 128), jnp.float32),
    mesh=_MESH,
    scratch_types=[
        pltpu.VMEM((IT, B), jnp.int32),
        pltpu.VMEM((B, 128), jnp.float32),
        pltpu.VMEM((RZ, 128), jnp.float32),
        pltpu.VMEM_SHARED((N_NODES, 128), jnp.float32),
    ])


def _inv_deg(dg):
  deg = dg[0, :, 0] + dg[1, :, 0]
  return (1.0 / jnp.maximum(deg, 1.0))[:, None]


def _dot(a, b):
  return jnp.dot(a, b, preferred_element_type=jnp.float32)


def _mm0_body(x0, x1, a0, a1, dg, ws, wn, b, o0, o1, o2, o3):
  inv = _inv_deg(dg[...])
  h = (_dot(x0[...], ws[0:128, :]) + _dot(x1[...], ws[128:256, :])
       + _dot((a0[0] + a0[1]) * inv, wn[0:128, :])
       + _dot((a1[0] + a1[1]) * inv, wn[128:256, :])
       + b[...][None, :])
  h = jnp.maximum(h, 0.0)
  o0[...] = h[:, 0:128]
  o1[...] = h[:, 128:256]
  o2[...] = h[:, 256:384]
  o3[...] = h[:, 384:512]


def _mm1_body(h0, h1, h2, h3, a0, a1, a2, a3, dg, ws, wn, b, wn2, oh, oz):
  inv = _inv_deg(dg[...])
  acc = b[...][None, :]
  hs = (h0, h1, h2, h3)
  As = (a0, a1, a2, a3)
  for c in range(4):
    acc = acc + _dot(hs[c][...], ws[c * 128:(c + 1) * 128, :])
    acc = acc + _dot((As[c][0] + As[c][1]) * inv, wn[c * 128:(c + 1) * 128, :])
  acc = jnp.maximum(acc, 0.0)
  oh[...] = acc
  z = _dot(acc, wn2[...])
  oz[...] = jnp.concatenate(
      [z, jnp.zeros((z.shape[0], 64), jnp.float32)], axis=1)


def _mm2_body(h, az, dg, ws, b, out):
  inv = _inv_deg(dg[...])
  out[...] = (_dot(h[...], ws[...]) + (az[0, :, 0:64] + az[1, :, 0:64]) * inv
              + b[...][None, :])


def _blk(shape, imap):
  return pl.BlockSpec(shape, imap)


_row = lambda i: (i, 0)
_part = lambda i: (0, i, 0)
_whole2 = lambda i: (0, 0)
_whole1 = lambda i: (0,)


def _mm0(x0, x1, a0, a1, dg, ws, wn, b):
  return pl.pallas_call(
      _mm0_body,
      grid=(G,),
      in_specs=[
          _blk((M_BLK, 128), _row), _blk((M_BLK, 128), _row),
          _blk((NC, M_BLK, 128), _part), _blk((NC, M_BLK, 128), _part),
          _blk((NC, M_BLK, 128), _part),
          _blk((256, 512), _whole2), _blk((256, 512), _whole2),
          _blk((512,), _whole1),
      ],
      out_specs=[_blk((M_BLK, 128), _row)] * 4,
      out_shape=[jax.ShapeDtypeStruct((N_NODES, 128), jnp.float32)] * 4,
  )(x0, x1, a0, a1, dg, ws, wn, b)


def _mm1(hs, As, dg, ws, wn, b, wn2):
  return pl.pallas_call(
      _mm1_body,
      grid=(G,),
      in_specs=(
          [_blk((M_BLK, 128), _row)] * 4
          + [_blk((NC, M_BLK, 128), _part)] * 4
          + [_blk((NC, M_BLK, 128), _part),
             _blk((512, 512), _whole2), _blk((512, 512), _whole2),
             _blk((512,), _whole1), _blk((512, 64), _whole2)]),
      out_specs=[_blk((M_BLK, 512), _row), _blk((M_BLK, 128), _row)],
      out_shape=[jax.ShapeDtypeStruct((N_NODES, 512), jnp.float32),
                 jax.ShapeDtypeStruct((N_NODES, 128), jnp.float32)],
  )(*hs, *As, dg, ws, wn, b, wn2)


def _mm2(h, az, dg, ws, b):
  return pl.pallas_call(
      _mm2_body,
      grid=(G,),
      in_specs=[
          _blk((M_BLK, 512), _row), _blk((NC, M_BLK, 128), _part),
          _blk((NC, M_BLK, 128), _part),
          _blk((512, 64), _whole2), _blk((64,), _whole1),
      ],
      out_specs=_blk((M_BLK, 64), _row),
      out_shape=jax.ShapeDtypeStruct((N_NODES, 64), jnp.float32),
  )(h, az, dg, ws, b)


_sc_agg2 = _make_sc_agg(2, 128)
_sc_agg4 = _make_sc_agg(4, 128)
_sc_agg1 = _make_sc_agg(1, 128)


@jax.jit
def kernel(x, edge_index, Ws0, Wn0, b0, Ws1, Wn1, b1, Ws2, Wn2, b2):
  ei = edge_index.astype(jnp.int32)
  src = ei[0].reshape(NW, IT, B)
  dst = ei[1].reshape(NW, IT, B)

  x0 = x[:, 0:128]
  x1 = x[:, 128:256]

  dg = _sc_deg(dst)
  a0, a1 = _sc_agg2(x0, x1, src, dst)
  hs = _mm0(x0, x1, a0, a1, dg, Ws0, Wn0, b0)
  As = _sc_agg4(*hs, src, dst)
  h2, z = _mm1(hs, As, dg, Ws1, Wn1, b1, Wn2)
  (az,) = _sc_agg1(z, src, dst)
  return _mm2(h2, az, dg, Ws2, b2)


# speedup vs baseline: 7.9909x; 1.0004x over previous
import functools

import jax
import jax.numpy as jnp
from jax import lax
from jax.experimental import pallas as pl
from jax.experimental.pallas import tpu as pltpu
from jax.experimental.pallas import tpu_sc as plsc

N_NODES = 10000
N_EDGES = 160000
IN_FEATS = 256
N_HIDDEN = 512
N_CLASSES = 64

NC = 2
NS = 16
NW = NC * NS
EPW = N_EDGES // NW
B = 100
IT = EPW // B
ND = 2
RPT = 624
RZ = 48
REM = N_NODES - NS * RPT

M_BLK = 1000
G = N_NODES // M_BLK


_MESH = plsc.VectorSubcoreMesh(core_axis_name="c", subcore_axis_name="s")
_REM0 = NS * RPT


def _make_sc_agg(n_chunks, feat):
  per_row = feat // 16

  out_type = [jax.ShapeDtypeStruct((NC, N_NODES, feat), jnp.float32)
              for _ in range(n_chunks)]
  scratch = [
      pltpu.VMEM((IT, B), jnp.int32),
      pltpu.VMEM((IT, B), jnp.int32),
  ] + [pltpu.VMEM((B, feat), jnp.float32) for _ in range(ND)] + [
      pltpu.VMEM((RZ, feat), jnp.float32),
      pltpu.VMEM_SHARED((N_NODES, feat), jnp.float32),
  ] + [pltpu.SemaphoreType.DMA for _ in range(ND)]

  def body(*refs):
    x_refs = refs[:n_chunks]
    src_hbm, dst_hbm = refs[n_chunks], refs[n_chunks + 1]
    o = n_chunks + 2
    out_refs = refs[o:o + n_chunks]
    rest = refs[o + n_chunks:]
    src_v, dst_v = rest[0], rest[1]
    rows = rest[2:2 + ND]
    zbuf, acc = rest[2 + ND], rest[3 + ND]
    sems = rest[4 + ND:4 + 2 * ND]

    cid = lax.axis_index("c")
    sid = lax.axis_index("s")
    wid = sid * NC + cid

    pltpu.sync_copy(src_hbm.at[wid], src_v)
    pltpu.sync_copy(dst_hbm.at[wid], dst_v)

    @pl.loop(0, RZ * per_row)
    def _fill_z(i):
      zbuf[i // per_row, pl.ds((i % per_row) * 16, 16)] = jnp.zeros(
          (16,), jnp.float32)

    row0 = sid * RPT

    def process_chunk(x_ref):
      for k in range(ND):
        pltpu.async_copy(x_ref.at[src_v.at[k]], rows[k], sems[k])

      @pl.loop(0, IT, step=ND)
      def _edges(j):
        for k in range(ND):
          pltpu.make_async_copy(
              x_ref.at[src_v.at[j + k]], rows[k], sems[k]).wait()
          pltpu.sync_copy(rows[k], acc.at[dst_v.at[j + k]], add=True)

          @pl.when(j + ND + k < IT)
          def _():
            pltpu.async_copy(x_ref.at[src_v.at[j + ND + k]], rows[k], sems[k])

    for ci in range(n_chunks):
      for k in range(RPT // RZ):
        pltpu.sync_copy(zbuf, acc.at[pl.ds(row0 + k * RZ, RZ)])

      @pl.when(sid == NS - 1)
      def _zero_rem():
        pltpu.sync_copy(zbuf.at[pl.ds(0, REM)], acc.at[pl.ds(_REM0, REM)])

      plsc.subcore_barrier()
      process_chunk(x_refs[ci])
      plsc.subcore_barrier()
      pltpu.sync_copy(acc.at[pl.ds(row0, RPT)],
                      out_refs[ci].at[cid, pl.ds(row0, RPT)])

      @pl.when(sid == NS - 1)
      def _out_rem():
        pltpu.sync_copy(acc.at[pl.ds(_REM0, REM)],
                        out_refs[ci].at[cid, pl.ds(_REM0, REM)])

      if ci + 1 < n_chunks:
        plsc.subcore_barrier()

  return pl.kernel(body, out_type=out_type, mesh=_MESH, scratch_types=scratch)


def _sc_deg_body(dst_hbm, deg_out, dst_v, ones_v, zdeg, degacc):
  cid = lax.axis_index("c")
  sid = lax.axis_index("s")
  wid = sid * NC + cid

  pltpu.sync_copy(dst_hbm.at[wid], dst_v)

  @pl.loop(0, B * 8)
  def _fill_o(i):
    ones_v[i // 8, pl.ds((i % 8) * 16, 16)] = jnp.full((16,), 1.0, jnp.float32)

  @pl.loop(0, RZ * 8)
  def _fill_zd(i):
    zdeg[i // 8, pl.ds((i % 8) * 16, 16)] = jnp.zeros((16,), jnp.float32)

  row0 = sid * RPT
  for k in range(RPT // RZ):
    pltpu.sync_copy(zdeg, degacc.at[pl.ds(row0 + k * RZ, RZ)])

  @pl.when(sid == NS - 1)
  def _zero_rem():
    pltpu.sync_copy(zdeg.at[pl.ds(0, REM)], degacc.at[pl.ds(_REM0, REM)])

  plsc.subcore_barrier()

  @pl.loop(0, IT)
  def _edges(j):
    pltpu.sync_copy(ones_v, degacc.at[dst_v.at[j]], add=True)

  plsc.subcore_barrier()
  pltpu.sync_copy(degacc.at[pl.ds(row0, RPT)],
                  deg_out.at[cid, pl.ds(row0, RPT)])

  @pl.when(sid == NS - 1)
  def _out_rem():
    pltpu.sync_copy(degacc.at[pl.ds(_REM0, REM)],
                    deg_out.at[cid, pl.ds(_REM0, REM)])


_sc_deg = pl.kernel(
    _sc_deg_body,
    out_type=jax.ShapeDtypeStruct((NC, N_NODES, 128), jnp.float32),
    mesh=_MESH,
    scratch_types=[
        pltpu.VMEM((IT, B), jnp.int32),
        pltpu.VMEM((B, 128), jnp.float32),
        pltpu.VMEM((RZ, 128), jnp.float32),
        pltpu.VMEM_SHARED((N_NODES, 128), jnp.float32),
    ])


def _inv_deg(dg):
  deg = dg[0, :, 0] + dg[1, :, 0]
  return (1.0 / jnp.maximum(deg, 1.0))[:, None]


def _dot(a, b):
  return jnp.dot(a, b, preferred_element_type=jnp.float32)


def _mm0_body(x0, x1, a0, a1, dg, ws, wn, b, o0, o1, o2, o3):
  inv = _inv_deg(dg[...])
  h = (_dot(x0[...], ws[0:128, :]) + _dot(x1[...], ws[128:256, :])
       + _dot((a0[0] + a0[1]) * inv, wn[0:128, :])
       + _dot((a1[0] + a1[1]) * inv, wn[128:256, :])
       + b[...][None, :])
  h = jnp.maximum(h, 0.0)
  o0[...] = h[:, 0:128]
  o1[...] = h[:, 128:256]
  o2[...] = h[:, 256:384]
  o3[...] = h[:, 384:512]


def _mm1_body(h0, h1, h2, h3, a0, a1, a2, a3, dg, ws, wn, b, wn2, oh, oz):
  inv = _inv_deg(dg[...])
  acc = b[...][None, :]
  hs = (h0, h1, h2, h3)
  As = (a0, a1, a2, a3)
  for c in range(4):
    acc = acc + _dot(hs[c][...], ws[c * 128:(c + 1) * 128, :])
    acc = acc + _dot((As[c][0] + As[c][1]) * inv, wn[c * 128:(c + 1) * 128, :])
  acc = jnp.maximum(acc, 0.0)
  oh[...] = acc
  z = _dot(acc, wn2[...])
  oz[...] = jnp.concatenate(
      [z, jnp.zeros((z.shape[0], 64), jnp.float32)], axis=1)


def _mm2_body(h, az, dg, ws, b, out):
  inv = _inv_deg(dg[...])
  out[...] = (_dot(h[...], ws[...]) + (az[0, :, 0:64] + az[1, :, 0:64]) * inv
              + b[...][None, :])


def _blk(shape, imap):
  return pl.BlockSpec(shape, imap)


_row = lambda i: (i, 0)
_part = lambda i: (0, i, 0)
_whole2 = lambda i: (0, 0)
_whole1 = lambda i: (0,)


def _mm0(x0, x1, a0, a1, dg, ws, wn, b):
  return pl.pallas_call(
      _mm0_body,
      grid=(G,),
      in_specs=[
          _blk((M_BLK, 128), _row), _blk((M_BLK, 128), _row),
          _blk((NC, M_BLK, 128), _part), _blk((NC, M_BLK, 128), _part),
          _blk((NC, M_BLK, 128), _part),
          _blk((256, 512), _whole2), _blk((256, 512), _whole2),
          _blk((512,), _whole1),
      ],
      out_specs=[_blk((M_BLK, 128), _row)] * 4,
      out_shape=[jax.ShapeDtypeStruct((N_NODES, 128), jnp.float32)] * 4,
  )(x0, x1, a0, a1, dg, ws, wn, b)


def _mm1(hs, As, dg, ws, wn, b, wn2):
  return pl.pallas_call(
      _mm1_body,
      grid=(G,),
      in_specs=(
          [_blk((M_BLK, 128), _row)] * 4
          + [_blk((NC, M_BLK, 128), _part)] * 4
          + [_blk((NC, M_BLK, 128), _part),
             _blk((512, 512), _whole2), _blk((512, 512), _whole2),
             _blk((512,), _whole1), _blk((512, 64), _whole2)]),
      out_specs=[_blk((M_BLK, 512), _row), _blk((M_BLK, 128), _row)],
      out_shape=[jax.ShapeDtypeStruct((N_NODES, 512), jnp.float32),
                 jax.ShapeDtypeStruct((N_NODES, 128), jnp.float32)],
  )(*hs, *As, dg, ws, wn, b, wn2)


def _mm2(h, az, dg, ws, b):
  return pl.pallas_call(
      _mm2_body,
      grid=(G,),
      in_specs=[
          _blk((M_BLK, 512), _row), _blk((NC, M_BLK, 128), _part),
          _blk((NC, M_BLK, 128), _part),
          _blk((512, 64), _whole2), _blk((64,), _whole1),
      ],
      out_specs=_blk((M_BLK, 64), _row),
      out_shape=jax.ShapeDtypeStruct((N_NODES, 64), jnp.float32),
  )(h, az, dg, ws, b)


_sc_agg2 = _make_sc_agg(2, 128)
_sc_agg4 = _make_sc_agg(4, 128)
_sc_agg1 = _make_sc_agg(1, 128)


@jax.jit
def kernel(x, edge_index, Ws0, Wn0, b0, Ws1, Wn1, b1, Ws2, Wn2, b2):
  ei = edge_index.astype(jnp.int32)
  src = ei[0].reshape(NW, IT, B)
  dst = ei[1].reshape(NW, IT, B)

  x0 = x[:, 0:128]
  x1 = x[:, 128:256]

  dg = _sc_deg(dst)
  a0, a1 = _sc_agg2(x0, x1, src, dst)
  hs = _mm0(x0, x1, a0, a1, dg, Ws0, Wn0, b0)
  As = _sc_agg4(*hs, src, dst)
  h2, z = _mm1(hs, As, dg, Ws1, Wn1, b1, Wn2)
  (az,) = _sc_agg1(z, src, dst)
  return _mm2(h2, az, dg, Ws2, b2)
